# initial kernel scaffold (unmeasured)
import jax
import jax.numpy as jnp
from jax import lax
from jax.experimental import pallas as pl
from jax.experimental.pallas import tpu as pltpu

NZ = 4
TL = 256
D = 1024
F = 2048
EL = 4
E = 16


def kernel(x, router, W1, W2):
    W1b = W1.astype(jnp.bfloat16)
    W2b = W2.astype(jnp.bfloat16)

    def body(x_ref, r_ref, w1_ref, w2_ref, out_ref,
             xcomm, rcomm, icomm, rsbuf, rrbuf, partial,
             xs_sems, xr_sems, rs_sems, rr_sems, is_sems, ir_sems,
             sss_sems, ssr_sems):
        ix = lax.axis_index("x")
        iy = lax.axis_index("y")
        iz = lax.axis_index("z")
        right = lax.rem(iz + 1, NZ)
        left = lax.rem(iz + NZ - 1, NZ)

        barrier = pltpu.get_barrier_semaphore()
        for nbr in (left, right):
            pl.semaphore_signal(
                barrier, inc=1,
                device_id=(ix, iy, nbr),
                device_id_type=pl.DeviceIdType.MESH,
            )
        pl.semaphore_wait(barrier, 2)

        rcomm[0] = r_ref[...]
        for h in range(NZ - 1):
            rd = pltpu.make_async_remote_copy(
                src_ref=rcomm.at[h],
                dst_ref=rcomm.at[h + 1],
                send_sem=rs_sems.at[h],
                recv_sem=rr_sems.at[h],
                device_id=(ix, iy, right),
                device_id_type=pl.DeviceIdType.MESH,
            )
            rd.start()
            rd.wait()

        xf = x_ref[...]
        R = jnp.concatenate([rcomm[s] for s in range(NZ)], axis=1)
        g = jnp.dot(xf, R, preferred_element_type=jnp.float32)

        iot = lax.broadcasted_iota(jnp.int32, (TL, E), 1)
        m1 = jnp.max(g, axis=1, keepdims=True)
        r1 = jnp.min(jnp.where(g >= m1, iot, E), axis=1, keepdims=True)
        g2 = jnp.where(iot == r1, -1e30, g)
        m2 = jnp.max(g2, axis=1, keepdims=True)
        r2 = jnp.min(jnp.where(g2 >= m2, iot, E), axis=1, keepdims=True)
        a1 = EL * lax.rem(iz - r1 // EL + NZ, NZ) + lax.rem(r1, EL)
        a2 = EL * lax.rem(iz - r2 // EL + NZ, NZ) + lax.rem(r2, EL)
        e2 = jnp.exp(m2 - m1)
        w1w = 1.0 / (1.0 + e2)
        w2w = e2 / (1.0 + e2)
        info = jnp.concatenate(
            [a1.astype(jnp.float32), a2.astype(jnp.float32), w1w, w2w], axis=1
        )

        xcomm[0] = xf.astype(jnp.bfloat16)
        icomm[0] = info
        for h in range(NZ - 1):
            rdx = pltpu.make_async_remote_copy(
                src_ref=xcomm.at[h],
                dst_ref=xcomm.at[h + 1],
                send_sem=xs_sems.at[h],
                recv_sem=xr_sems.at[h],
                device_id=(ix, iy, right),
                device_id_type=pl.DeviceIdType.MESH,
            )
            rdi = pltpu.make_async_remote_copy(
                src_ref=icomm.at[h],
                dst_ref=icomm.at[h + 1],
                send_sem=is_sems.at[h],
                recv_sem=ir_sems.at[h],
                device_id=(ix, iy, right),
                device_id_type=pl.DeviceIdType.MESH,
            )
            rdx.start()
            rdi.start()
            rdx.wait()
            rdi.wait()

        X = jnp.concatenate([xcomm[s] for s in range(NZ)], axis=0)
        A1 = jnp.concatenate([icomm[s][:, 0:1] for s in range(NZ)], axis=0)
        A2 = jnp.concatenate([icomm[s][:, 1:2] for s in range(NZ)], axis=0)
        Wa = jnp.concatenate([icomm[s][:, 2:3] for s in range(NZ)], axis=0)
        Wb = jnp.concatenate([icomm[s][:, 3:4] for s in range(NZ)], axis=0)

        acc = jnp.zeros((NZ * TL, D), jnp.float32)
        for c in range(EL):
            eg = (EL * iz + c).astype(jnp.float32)
            sel = (A1 == eg).astype(jnp.float32) * Wa \
                + (A2 == eg).astype(jnp.float32) * Wb
            h1 = jnp.maximum(
                jnp.dot(X, w1_ref[c], preferred_element_type=jnp.float32), 0.0
            )
            y = jnp.dot(
                h1.astype(jnp.bfloat16), w2_ref[c],
                preferred_element_type=jnp.float32,
            )
            acc = acc + y * sel
        partial[...] = acc

        for s in range(NZ - 1):
            val = partial[pl.ds((s + 1) * TL, TL), :]
            if s > 0:
                val = val + rrbuf[s - 1].astype(jnp.float32)
            rsbuf[s] = val.astype(jnp.bfloat16)
            rd = pltpu.make_async_remote_copy(
                src_ref=rsbuf.at[s],
                dst_ref=rrbuf.at[s],
                send_sem=sss_sems.at[s],
                recv_sem=ssr_sems.at[s],
                device_id=(ix, iy, right),
                device_id_type=pl.DeviceIdType.MESH,
            )
            rd.start()
            rd.wait()
        out_ref[...] = (
            partial[pl.ds(0, TL), :] + rrbuf[NZ - 2].astype(jnp.float32)
        )

    return pl.pallas_call(
        body,
        out_shape=jax.ShapeDtypeStruct((TL, D), jnp.float32),
        in_specs=[
            pl.BlockSpec(memory_space=pltpu.VMEM),
            pl.BlockSpec(memory_space=pltpu.VMEM),
            pl.BlockSpec(memory_space=pltpu.VMEM),
            pl.BlockSpec(memory_space=pltpu.VMEM),
        ],
        out_specs=pl.BlockSpec(memory_space=pltpu.VMEM),
        scratch_shapes=[
            pltpu.VMEM((NZ, TL, D), jnp.bfloat16),
            pltpu.VMEM((NZ, D, EL), jnp.float32),
            pltpu.VMEM((NZ, TL, EL), jnp.float32),
            pltpu.VMEM((NZ - 1, TL, D), jnp.bfloat16),
            pltpu.VMEM((NZ - 1, TL, D), jnp.bfloat16),
            pltpu.VMEM((NZ * TL, D), jnp.float32),
            pltpu.SemaphoreType.DMA((NZ - 1,)),
            pltpu.SemaphoreType.DMA((NZ - 1,)),
            pltpu.SemaphoreType.DMA((NZ - 1,)),
            pltpu.SemaphoreType.DMA((NZ - 1,)),
            pltpu.SemaphoreType.DMA((NZ - 1,)),
            pltpu.SemaphoreType.DMA((NZ - 1,)),
            pltpu.SemaphoreType.DMA((NZ - 1,)),
            pltpu.SemaphoreType.DMA((NZ - 1,)),
        ],
        compiler_params=pltpu.CompilerParams(collective_id=0),
    )(x, router, W1b, W2b)


# baseline (device time: 167507 ns/iter reference)
import jax
import jax.numpy as jnp
from jax import lax
from jax.experimental import pallas as pl
from jax.experimental.pallas import tpu as pltpu

NZ = 4
TL = 256
D = 1024
F = 2048
EL = 4
E = 16


def kernel(x, router, W1, W2):
    W1b = W1.astype(jnp.bfloat16)
    W2b = W2.astype(jnp.bfloat16)

    def body(x_ref, r_ref, w1_ref, w2_ref, out_ref,
             xcomm, rcomm, icomm, rsbuf, rrbuf, partial,
             xs_sems, xr_sems, rs_sems, rr_sems, is_sems, ir_sems,
             sss_sems, ssr_sems):
        ix = lax.axis_index("x")
        iy = lax.axis_index("y")
        iz = lax.axis_index("z")
        right = lax.rem(iz + 1, NZ)
        left = lax.rem(iz + NZ - 1, NZ)

        barrier = pltpu.get_barrier_semaphore()
        for nbr in (left, right):
            pl.semaphore_signal(
                barrier, inc=1,
                device_id=(ix, iy, nbr),
                device_id_type=pl.DeviceIdType.MESH,
            )
        pl.semaphore_wait(barrier, 2)

        rcomm[0] = r_ref[...]
        for h in range(NZ - 1):
            rd = pltpu.make_async_remote_copy(
                src_ref=rcomm.at[h],
                dst_ref=rcomm.at[h + 1],
                send_sem=rs_sems.at[h],
                recv_sem=rr_sems.at[h],
                device_id=(ix, iy, right),
                device_id_type=pl.DeviceIdType.MESH,
            )
            rd.start()
            rd.wait()

        xf = x_ref[...]
        R = jnp.concatenate([rcomm[s] for s in range(NZ)], axis=1)
        g = jnp.dot(xf, R, preferred_element_type=jnp.float32,
                    precision=lax.Precision.HIGHEST)

        iot = lax.broadcasted_iota(jnp.int32, (TL, E), 1)
        m1 = jnp.max(g, axis=1, keepdims=True)
        r1 = jnp.min(jnp.where(g >= m1, iot, E), axis=1, keepdims=True)
        g2 = jnp.where(iot == r1, -1e30, g)
        m2 = jnp.max(g2, axis=1, keepdims=True)
        r2 = jnp.min(jnp.where(g2 >= m2, iot, E), axis=1, keepdims=True)
        a1 = EL * lax.rem(iz - r1 // EL + NZ, NZ) + lax.rem(r1, EL)
        a2 = EL * lax.rem(iz - r2 // EL + NZ, NZ) + lax.rem(r2, EL)
        e2 = jnp.exp(m2 - m1)
        w1w = 1.0 / (1.0 + e2)
        w2w = e2 / (1.0 + e2)
        info = jnp.concatenate(
            [a1.astype(jnp.float32), a2.astype(jnp.float32), w1w, w2w], axis=1
        )

        xcomm[0] = xf.astype(jnp.bfloat16)
        icomm[0] = info
        for h in range(NZ - 1):
            rdx = pltpu.make_async_remote_copy(
                src_ref=xcomm.at[h],
                dst_ref=xcomm.at[h + 1],
                send_sem=xs_sems.at[h],
                recv_sem=xr_sems.at[h],
                device_id=(ix, iy, right),
                device_id_type=pl.DeviceIdType.MESH,
            )
            rdi = pltpu.make_async_remote_copy(
                src_ref=icomm.at[h],
                dst_ref=icomm.at[h + 1],
                send_sem=is_sems.at[h],
                recv_sem=ir_sems.at[h],
                device_id=(ix, iy, right),
                device_id_type=pl.DeviceIdType.MESH,
            )
            rdx.start()
            rdi.start()
            rdx.wait()
            rdi.wait()

        X = jnp.concatenate([xcomm[s] for s in range(NZ)], axis=0)
        A1 = jnp.concatenate([icomm[s][:, 0:1] for s in range(NZ)], axis=0)
        A2 = jnp.concatenate([icomm[s][:, 1:2] for s in range(NZ)], axis=0)
        Wa = jnp.concatenate([icomm[s][:, 2:3] for s in range(NZ)], axis=0)
        Wb = jnp.concatenate([icomm[s][:, 3:4] for s in range(NZ)], axis=0)

        def expert_step(c, acc):
            eg = (EL * iz + c).astype(jnp.float32)
            sel = (A1 == eg).astype(jnp.float32) * Wa \
                + (A2 == eg).astype(jnp.float32) * Wb
            h1 = jnp.maximum(
                jnp.dot(X, w1_ref[c], preferred_element_type=jnp.float32), 0.0
            )
            y = jnp.dot(
                h1.astype(jnp.bfloat16), w2_ref[c],
                preferred_element_type=jnp.float32,
            )
            return acc + y * sel

        partial[...] = lax.fori_loop(
            0, EL, expert_step, jnp.zeros((NZ * TL, D), jnp.float32)
        )

        for s in range(NZ - 1):
            val = partial[pl.ds((s + 1) * TL, TL), :]
            if s > 0:
                val = val + rrbuf[s - 1].astype(jnp.float32)
            rsbuf[s] = val.astype(jnp.bfloat16)
            rd = pltpu.make_async_remote_copy(
                src_ref=rsbuf.at[s],
                dst_ref=rrbuf.at[s],
                send_sem=sss_sems.at[s],
                recv_sem=ssr_sems.at[s],
                device_id=(ix, iy, right),
                device_id_type=pl.DeviceIdType.MESH,
            )
            rd.start()
            rd.wait()
        out_ref[...] = (
            partial[pl.ds(0, TL), :] + rrbuf[NZ - 2].astype(jnp.float32)
        )

    return pl.pallas_call(
        body,
        out_shape=jax.ShapeDtypeStruct((TL, D), jnp.float32),
        in_specs=[
            pl.BlockSpec(memory_space=pltpu.VMEM),
            pl.BlockSpec(memory_space=pltpu.VMEM),
            pl.BlockSpec(memory_space=pltpu.VMEM),
            pl.BlockSpec(memory_space=pltpu.VMEM),
        ],
        out_specs=pl.BlockSpec(memory_space=pltpu.VMEM),
        scratch_shapes=[
            pltpu.VMEM((NZ, TL, D), jnp.bfloat16),
            pltpu.VMEM((NZ, D, EL), jnp.float32),
            pltpu.VMEM((NZ, TL, EL), jnp.float32),
            pltpu.VMEM((NZ - 1, TL, D), jnp.bfloat16),
            pltpu.VMEM((NZ - 1, TL, D), jnp.bfloat16),
            pltpu.VMEM((NZ * TL, D), jnp.float32),
            pltpu.SemaphoreType.DMA((NZ - 1,)),
            pltpu.SemaphoreType.DMA((NZ - 1,)),
            pltpu.SemaphoreType.DMA((NZ - 1,)),
            pltpu.SemaphoreType.DMA((NZ - 1,)),
            pltpu.SemaphoreType.DMA((NZ - 1,)),
            pltpu.SemaphoreType.DMA((NZ - 1,)),
            pltpu.SemaphoreType.DMA((NZ - 1,)),
            pltpu.SemaphoreType.DMA((NZ - 1,)),
        ],
        compiler_params=pltpu.CompilerParams(
            collective_id=0,
            vmem_limit_bytes=100 * 1024 * 1024,
        ),
    )(x, router, W1b, W2b)


# device time: 124689 ns/iter; 1.3434x vs baseline; 1.3434x over previous
import jax
import jax.numpy as jnp
from jax import lax
from jax.experimental import pallas as pl
from jax.experimental.pallas import tpu as pltpu

NZ = 4
TL = 256
D = 1024
F = 2048
EL = 4
E = 16


def kernel(x, router, W1, W2):
    W1b = W1.astype(jnp.bfloat16)
    W2b = W2.astype(jnp.bfloat16)

    def body(x_ref, r_ref, w1_ref, w2_ref, out_ref,
             xcomm, rcomm, icomm, rsbuf, rrbuf, partial, wst1, wst2,
             xs_sems, xr_sems, rs_sems, rr_sems, is_sems, ir_sems,
             sss_sems, ssr_sems, w1_sems, w2_sems):
        ix = lax.axis_index("x")
        iy = lax.axis_index("y")
        iz = lax.axis_index("z")
        right = lax.rem(iz + 1, NZ)
        left = lax.rem(iz + NZ - 1, NZ)

        w_copies = []
        for c in range(EL):
            cp1 = pltpu.make_async_copy(w1_ref.at[c], wst1.at[c], w1_sems.at[c])
            cp2 = pltpu.make_async_copy(w2_ref.at[c], wst2.at[c], w2_sems.at[c])
            cp1.start()
            cp2.start()
            w_copies.append((cp1, cp2))

        barrier = pltpu.get_barrier_semaphore()
        for nbr in (left, right):
            pl.semaphore_signal(
                barrier, inc=1,
                device_id=(ix, iy, nbr),
                device_id_type=pl.DeviceIdType.MESH,
            )
        pl.semaphore_wait(barrier, 2)

        rcomm[0] = r_ref[...]
        for h in range(NZ - 1):
            rd = pltpu.make_async_remote_copy(
                src_ref=rcomm.at[h],
                dst_ref=rcomm.at[h + 1],
                send_sem=rs_sems.at[h],
                recv_sem=rr_sems.at[h],
                device_id=(ix, iy, right),
                device_id_type=pl.DeviceIdType.MESH,
            )
            rd.start()
            rd.wait()

        xf = x_ref[...]
        R = jnp.concatenate([rcomm[s] for s in range(NZ)], axis=1)
        g = jnp.dot(xf, R, preferred_element_type=jnp.float32,
                    precision=lax.Precision.HIGHEST)

        iot = lax.broadcasted_iota(jnp.int32, (TL, E), 1)
        m1 = jnp.max(g, axis=1, keepdims=True)
        r1 = jnp.min(jnp.where(g >= m1, iot, E), axis=1, keepdims=True)
        g2 = jnp.where(iot == r1, -1e30, g)
        m2 = jnp.max(g2, axis=1, keepdims=True)
        r2 = jnp.min(jnp.where(g2 >= m2, iot, E), axis=1, keepdims=True)
        a1 = EL * lax.rem(iz - r1 // EL + NZ, NZ) + lax.rem(r1, EL)
        a2 = EL * lax.rem(iz - r2 // EL + NZ, NZ) + lax.rem(r2, EL)
        e2 = jnp.exp(m2 - m1)
        w1w = 1.0 / (1.0 + e2)
        w2w = e2 / (1.0 + e2)
        info = jnp.concatenate(
            [a1.astype(jnp.float32), a2.astype(jnp.float32), w1w, w2w], axis=1
        )

        xcomm[0] = xf.astype(jnp.bfloat16)
        icomm[0] = info

        for cp1, cp2 in w_copies:
            cp1.wait()
            cp2.wait()

        ag = []

        def start_ag(h):
            rdx = pltpu.make_async_remote_copy(
                src_ref=xcomm.at[h],
                dst_ref=xcomm.at[h + 1],
                send_sem=xs_sems.at[h],
                recv_sem=xr_sems.at[h],
                device_id=(ix, iy, right),
                device_id_type=pl.DeviceIdType.MESH,
            )
            rdi = pltpu.make_async_remote_copy(
                src_ref=icomm.at[h],
                dst_ref=icomm.at[h + 1],
                send_sem=is_sems.at[h],
                recv_sem=ir_sems.at[h],
                device_id=(ix, iy, right),
                device_id_type=pl.DeviceIdType.MESH,
            )
            rdx.start()
            rdi.start()
            ag.append((rdx, rdi))

        def ffn_slot(s):
            Xs = xcomm[s][...]
            A1 = icomm[s][:, 0:1]
            A2 = icomm[s][:, 1:2]
            Wa = icomm[s][:, 2:3]
            Wb = icomm[s][:, 3:4]

            def expert_step(c, acc):
                eg = (EL * iz + c).astype(jnp.float32)
                sel = (A1 == eg).astype(jnp.float32) * Wa \
                    + (A2 == eg).astype(jnp.float32) * Wb
                Xsel = Xs * sel.astype(jnp.bfloat16)
                h1 = jnp.maximum(
                    jnp.dot(Xsel, wst1[c], preferred_element_type=jnp.float32),
                    0.0,
                )
                y = jnp.dot(
                    h1.astype(jnp.bfloat16), wst2[c],
                    preferred_element_type=jnp.float32,
                )
                return acc + y

            partial[pl.ds(s * TL, TL), :] = lax.fori_loop(
                0, EL, expert_step, jnp.zeros((TL, D), jnp.float32)
            )

        rs = []

        def start_rs(s):
            val = partial[pl.ds((s + 1) * TL, TL), :]
            if s > 0:
                val = val + rrbuf[s - 1].astype(jnp.float32)
            rsbuf[s] = val.astype(jnp.bfloat16)
            rd = pltpu.make_async_remote_copy(
                src_ref=rsbuf.at[s],
                dst_ref=rrbuf.at[s],
                send_sem=sss_sems.at[s],
                recv_sem=ssr_sems.at[s],
                device_id=(ix, iy, right),
                device_id_type=pl.DeviceIdType.MESH,
            )
            rd.start()
            rs.append(rd)

        start_ag(0)
        ffn_slot(0)
        ag[0][0].wait()
        ag[0][1].wait()
        start_ag(1)
        ffn_slot(1)
        start_rs(0)
        ag[1][0].wait()
        ag[1][1].wait()
        start_ag(2)
        ffn_slot(2)
        rs[0].wait()
        start_rs(1)
        ag[2][0].wait()
        ag[2][1].wait()
        ffn_slot(3)
        rs[1].wait()
        start_rs(2)
        rs[2].wait()
        out_ref[...] = (
            partial[pl.ds(0, TL), :] + rrbuf[NZ - 2].astype(jnp.float32)
        )

    return pl.pallas_call(
        body,
        out_shape=jax.ShapeDtypeStruct((TL, D), jnp.float32),
        in_specs=[
            pl.BlockSpec(memory_space=pltpu.VMEM),
            pl.BlockSpec(memory_space=pltpu.VMEM),
            pl.BlockSpec(memory_space=pl.ANY),
            pl.BlockSpec(memory_space=pl.ANY),
        ],
        out_specs=pl.BlockSpec(memory_space=pltpu.VMEM),
        scratch_shapes=[
            pltpu.VMEM((NZ, TL, D), jnp.bfloat16),
            pltpu.VMEM((NZ, D, EL), jnp.float32),
            pltpu.VMEM((NZ, TL, EL), jnp.float32),
            pltpu.VMEM((NZ - 1, TL, D), jnp.bfloat16),
            pltpu.VMEM((NZ - 1, TL, D), jnp.bfloat16),
            pltpu.VMEM((NZ * TL, D), jnp.float32),
            pltpu.VMEM((EL, D, F), jnp.bfloat16),
            pltpu.VMEM((EL, F, D), jnp.bfloat16),
            pltpu.SemaphoreType.DMA((NZ - 1,)),
            pltpu.SemaphoreType.DMA((NZ - 1,)),
            pltpu.SemaphoreType.DMA((NZ - 1,)),
            pltpu.SemaphoreType.DMA((NZ - 1,)),
            pltpu.SemaphoreType.DMA((NZ - 1,)),
            pltpu.SemaphoreType.DMA((NZ - 1,)),
            pltpu.SemaphoreType.DMA((NZ - 1,)),
            pltpu.SemaphoreType.DMA((NZ - 1,)),
            pltpu.SemaphoreType.DMA((EL,)),
            pltpu.SemaphoreType.DMA((EL,)),
        ],
        compiler_params=pltpu.CompilerParams(
            collective_id=0,
            vmem_limit_bytes=60 * 1024 * 1024,
        ),
    )(x, router, W1b, W2b)


# device time: 106648 ns/iter; 1.5707x vs baseline; 1.1692x over previous
import jax
import jax.numpy as jnp
from jax import lax
from jax.experimental import pallas as pl
from jax.experimental.pallas import tpu as pltpu

NZ = 4
TL = 256
D = 1024
F = 2048
EL = 4
E = 16


def kernel(x, router, W1, W2):
    def body(x_ref, r_ref, w1_ref, w2_ref, out_ref,
             xcomm, rcomm, icomm, rsbuf, rrbuf, partial, wf1, wf2,
             xs_sems, xr_sems, rs_sems, rr_sems, is_sems, ir_sems,
             sss_sems, ssr_sems, w1_sems, w2_sems):
        ix = lax.axis_index("x")
        iy = lax.axis_index("y")
        iz = lax.axis_index("z")
        right = lax.rem(iz + 1, NZ)
        left = lax.rem(iz + NZ - 1, NZ)

        def stream_pair(p):
            out = []
            for j in range(2):
                c = 2 * p + j
                cp1 = pltpu.make_async_copy(
                    w1_ref.at[c], wf1.at[j], w1_sems.at[j])
                cp2 = pltpu.make_async_copy(
                    w2_ref.at[c], wf2.at[j], w2_sems.at[j])
                cp1.start()
                cp2.start()
                out += [cp1, cp2]
            return out

        pair_dmas = stream_pair(0)

        barrier = pltpu.get_barrier_semaphore()
        for nbr in (left, right):
            pl.semaphore_signal(
                barrier, inc=1,
                device_id=(ix, iy, nbr),
                device_id_type=pl.DeviceIdType.MESH,
            )
        pl.semaphore_wait(barrier, 2)

        rcomm[0] = r_ref[...]
        for h in range(NZ - 1):
            rd = pltpu.make_async_remote_copy(
                src_ref=rcomm.at[h],
                dst_ref=rcomm.at[h + 1],
                send_sem=rs_sems.at[h],
                recv_sem=rr_sems.at[h],
                device_id=(ix, iy, right),
                device_id_type=pl.DeviceIdType.MESH,
            )
            rd.start()
            rd.wait()

        xf = x_ref[...]
        R = jnp.concatenate([rcomm[s] for s in range(NZ)], axis=1)
        g = jnp.dot(xf, R, preferred_element_type=jnp.float32,
                    precision=lax.Precision.HIGHEST)

        iot = lax.broadcasted_iota(jnp.int32, (TL, E), 1)
        m1 = jnp.max(g, axis=1, keepdims=True)
        r1 = jnp.min(jnp.where(g >= m1, iot, E), axis=1, keepdims=True)
        g2 = jnp.where(iot == r1, -1e30, g)
        m2 = jnp.max(g2, axis=1, keepdims=True)
        r2 = jnp.min(jnp.where(g2 >= m2, iot, E), axis=1, keepdims=True)
        a1 = EL * lax.rem(iz - r1 // EL + NZ, NZ) + lax.rem(r1, EL)
        a2 = EL * lax.rem(iz - r2 // EL + NZ, NZ) + lax.rem(r2, EL)
        e2 = jnp.exp(m2 - m1)
        w1w = 1.0 / (1.0 + e2)
        w2w = e2 / (1.0 + e2)
        info = jnp.concatenate(
            [a1.astype(jnp.float32), a2.astype(jnp.float32), w1w, w2w], axis=1
        )

        xcomm[0] = xf.astype(jnp.bfloat16)
        icomm[0] = info

        ag = []

        def start_ag(h):
            rdx = pltpu.make_async_remote_copy(
                src_ref=xcomm.at[h],
                dst_ref=xcomm.at[h + 1],
                send_sem=xs_sems.at[h],
                recv_sem=xr_sems.at[h],
                device_id=(ix, iy, right),
                device_id_type=pl.DeviceIdType.MESH,
            )
            rdi = pltpu.make_async_remote_copy(
                src_ref=icomm.at[h],
                dst_ref=icomm.at[h + 1],
                send_sem=is_sems.at[h],
                recv_sem=ir_sems.at[h],
                device_id=(ix, iy, right),
                device_id_type=pl.DeviceIdType.MESH,
            )
            rdx.start()
            rdi.start()
            ag.append((rdx, rdi))

        def wait_ag(h):
            ag[h][0].wait()
            ag[h][1].wait()

        def pair_compute(p, s):
            Xs = xcomm[s][...].astype(jnp.float32)
            A1 = icomm[s][:, 0:1]
            A2 = icomm[s][:, 1:2]
            Wa = icomm[s][:, 2:3]
            Wb = icomm[s][:, 3:4]

            def expert_step(j, acc):
                eg = (EL * iz + 2 * p + j).astype(jnp.float32)
                sel = (A1 == eg).astype(jnp.float32) * Wa \
                    + (A2 == eg).astype(jnp.float32) * Wb
                Xsel = Xs * sel
                h1 = jnp.maximum(
                    jnp.dot(Xsel, wf1[j], preferred_element_type=jnp.float32),
                    0.0,
                )
                y = jnp.dot(h1, wf2[j], preferred_element_type=jnp.float32)
                return acc + y

            return lax.fori_loop(
                0, 2, expert_step, jnp.zeros((TL, D), jnp.float32)
            )

        for cp in pair_dmas:
            cp.wait()
        start_ag(0)
        partial[pl.ds(0, TL), :] = pair_compute(0, 0)
        wait_ag(0)
        start_ag(1)
        partial[pl.ds(TL, TL), :] = pair_compute(0, 1)
        wait_ag(1)
        start_ag(2)
        partial[pl.ds(2 * TL, TL), :] = pair_compute(0, 2)
        wait_ag(2)
        partial[pl.ds(3 * TL, TL), :] = pair_compute(0, 3)

        pair_dmas = stream_pair(1)
        for cp in pair_dmas:
            cp.wait()

        rs = []

        def start_rs(s, res):
            val = partial[pl.ds((s + 1) * TL, TL), :] + res
            if s > 0:
                val = val + rrbuf[s - 1].astype(jnp.float32)
            rsbuf[s] = val.astype(jnp.bfloat16)
            rd = pltpu.make_async_remote_copy(
                src_ref=rsbuf.at[s],
                dst_ref=rrbuf.at[s],
                send_sem=sss_sems.at[s],
                recv_sem=ssr_sems.at[s],
                device_id=(ix, iy, right),
                device_id_type=pl.DeviceIdType.MESH,
            )
            rd.start()
            rs.append(rd)

        start_rs(0, pair_compute(1, 1))
        r2v = pair_compute(1, 2)
        rs[0].wait()
        start_rs(1, r2v)
        r3v = pair_compute(1, 3)
        rs[1].wait()
        start_rs(2, r3v)
        own = partial[pl.ds(0, TL), :] + pair_compute(1, 0)
        rs[2].wait()
        out_ref[...] = own + rrbuf[NZ - 2].astype(jnp.float32)

    return pl.pallas_call(
        body,
        out_shape=jax.ShapeDtypeStruct((TL, D), jnp.float32),
        in_specs=[
            pl.BlockSpec(memory_space=pltpu.VMEM),
            pl.BlockSpec(memory_space=pltpu.VMEM),
            pl.BlockSpec(memory_space=pl.ANY),
            pl.BlockSpec(memory_space=pl.ANY),
        ],
        out_specs=pl.BlockSpec(memory_space=pltpu.VMEM),
        scratch_shapes=[
            pltpu.VMEM((NZ, TL, D), jnp.bfloat16),
            pltpu.VMEM((NZ, D, EL), jnp.float32),
            pltpu.VMEM((NZ, TL, EL), jnp.float32),
            pltpu.VMEM((NZ - 1, TL, D), jnp.bfloat16),
            pltpu.VMEM((NZ - 1, TL, D), jnp.bfloat16),
            pltpu.VMEM((NZ * TL, D), jnp.float32),
            pltpu.VMEM((2, D, F), jnp.float32),
            pltpu.VMEM((2, F, D), jnp.float32),
            pltpu.SemaphoreType.DMA((NZ - 1,)),
            pltpu.SemaphoreType.DMA((NZ - 1,)),
            pltpu.SemaphoreType.DMA((NZ - 1,)),
            pltpu.SemaphoreType.DMA((NZ - 1,)),
            pltpu.SemaphoreType.DMA((NZ - 1,)),
            pltpu.SemaphoreType.DMA((NZ - 1,)),
            pltpu.SemaphoreType.DMA((NZ - 1,)),
            pltpu.SemaphoreType.DMA((NZ - 1,)),
            pltpu.SemaphoreType.DMA((2,)),
            pltpu.SemaphoreType.DMA((2,)),
        ],
        compiler_params=pltpu.CompilerParams(
            collective_id=0,
            vmem_limit_bytes=63 * 1024 * 1024,
        ),
    )(x, router, W1, W2)
